# baseline (speedup 1.0000x reference)
import functools
import jax
import jax.numpy as jnp
from jax import lax
from jax.experimental import pallas as pl
from jax.experimental.pallas import tpu as pltpu, tpu_sc as plsc

NUM_PATCHES = 576
LATENT_DIM = 768
BATCH = 64

NC = 2
NS = 16
NW = NC * NS
PP = NUM_PATCHES // NW
NLANE = LATENT_DIM // 16

_mesh = plsc.VectorSubcoreMesh(
    core_axis_name="c", subcore_axis_name="s", num_cores=NC, num_subcores=NS)


@functools.partial(
    pl.kernel,
    out_type=jax.ShapeDtypeStruct((BATCH, NUM_PATCHES, LATENT_DIM), jnp.float32),
    mesh=_mesh,
    scratch_types=[
        pltpu.VMEM((PP, LATENT_DIM), jnp.float32),
        pltpu.VMEM((PP, LATENT_DIM), jnp.float32),
        pltpu.VMEM((PP, LATENT_DIM), jnp.float32),
    ],
    compiler_params=pltpu.CompilerParams(use_tc_tiling_on_sc=False),
)
def _sc_add(x_hbm, pos_hbm, out_hbm, pbuf, xbuf, obuf):
    wid = lax.axis_index("s") * NC + lax.axis_index("c")
    p0 = wid * PP
    pltpu.sync_copy(pos_hbm.at[pl.ds(p0, PP)], pbuf)

    def batch_body(b, carry):
        pltpu.sync_copy(x_hbm.at[b, pl.ds(p0, PP)], xbuf)

        def row_body(r, carry2):
            def chunk_body(c, carry3):
                off = c * 16
                obuf[r, pl.ds(off, 16)] = (
                    xbuf[r, pl.ds(off, 16)] + pbuf[r, pl.ds(off, 16)])
                return carry3
            return lax.fori_loop(0, NLANE, chunk_body, carry2)

        lax.fori_loop(0, PP, row_body, 0)
        pltpu.sync_copy(obuf, out_hbm.at[b, pl.ds(p0, PP)])
        return carry

    lax.fori_loop(0, BATCH, batch_body, 0)


def kernel(x, pos_table):
    return _sc_add(x, pos_table)


# speedup vs baseline: 10.1009x; 10.1009x over previous
import jax
import jax.numpy as jnp
from jax.experimental import pallas as pl
from jax.experimental.pallas import tpu as pltpu

NUM_PATCHES = 576
LATENT_DIM = 768
BATCH = 64

SIZES = (2, 2, 4, 8, 8, 8, 8, 8, 8, 4, 2, 2)
STARTS = tuple(sum(SIZES[:i]) for i in range(len(SIZES)))
MAXB = max(SIZES)
NSTEP = len(SIZES)


def _pipeline(x_hbm, pos_ref, out_hbm, xbuf, obuf, in_sem, out_sem):
    def in_copy(k):
        s, b0, bb = k % 2, STARTS[k], SIZES[k]
        return pltpu.make_async_copy(
            x_hbm.at[pl.ds(b0, bb)], xbuf.at[s, pl.ds(0, bb)], in_sem.at[s])

    def out_copy(k):
        s, b0, bb = k % 2, STARTS[k], SIZES[k]
        return pltpu.make_async_copy(
            obuf.at[s, pl.ds(0, bb)], out_hbm.at[pl.ds(b0, bb)], out_sem.at[s])

    in_copy(0).start()
    in_copy(1).start()

    for k in range(NSTEP):
        s, bb = k % 2, SIZES[k]
        in_copy(k).wait()
        if k >= 2:
            out_copy(k - 2).wait()
        obuf[s, pl.ds(0, bb)] = xbuf[s, pl.ds(0, bb)] + pos_ref[...]
        out_copy(k).start()
        if k + 2 < NSTEP:
            in_copy(k + 2).start()

    out_copy(NSTEP - 2).wait()
    out_copy(NSTEP - 1).wait()


def kernel(x, pos_table):
    return pl.pallas_call(
        _pipeline,
        in_specs=[
            pl.BlockSpec(memory_space=pltpu.HBM),
            pl.BlockSpec(memory_space=pltpu.VMEM),
        ],
        out_specs=pl.BlockSpec(memory_space=pltpu.HBM),
        out_shape=jax.ShapeDtypeStruct((BATCH, NUM_PATCHES, LATENT_DIM), x.dtype),
        scratch_shapes=[
            pltpu.VMEM((2, MAXB, NUM_PATCHES, LATENT_DIM), jnp.float32),
            pltpu.VMEM((2, MAXB, NUM_PATCHES, LATENT_DIM), jnp.float32),
            pltpu.SemaphoreType.DMA((2,)),
            pltpu.SemaphoreType.DMA((2,)),
        ],
    )(x, pos_table)


# speedup vs baseline: 10.3109x; 1.0208x over previous
import jax
import jax.numpy as jnp
from jax.experimental import pallas as pl

NUM_PATCHES = 576
LATENT_DIM = 768
BATCH = 64

BB = 8


def _add_kernel(x_ref, pos_ref, out_ref):
    out_ref[...] = x_ref[...] + pos_ref[...]


def kernel(x, pos_table):
    return pl.pallas_call(
        _add_kernel,
        grid=(BATCH // BB,),
        in_specs=[
            pl.BlockSpec((BB, NUM_PATCHES, LATENT_DIM), lambda b: (b, 0, 0)),
            pl.BlockSpec((NUM_PATCHES, LATENT_DIM), lambda b: (0, 0)),
        ],
        out_specs=pl.BlockSpec((BB, NUM_PATCHES, LATENT_DIM), lambda b: (b, 0, 0)),
        out_shape=jax.ShapeDtypeStruct((BATCH, NUM_PATCHES, LATENT_DIM), x.dtype),
    )(x, pos_table)
